# baseline (speedup 1.0000x reference)
import functools

import jax
import jax.numpy as jnp
from jax import lax
from jax.experimental import pallas as pl
from jax.experimental.pallas import tpu as pltpu
from jax.experimental.pallas import tpu_sc as plsc

OUT_DIM = 64
N_CTRL = 16
BATCH = 1024
WIDTH = OUT_DIM * N_CTRL

_INFO = plsc.get_sparse_core_info()
_NC = 1
_NS = _INFO.num_subcores
_NW = _NC * _NS
_L = _INFO.num_lanes
_B_PER_W = BATCH // _NW
_N_CHUNK = _B_PER_W // _L


def _mux_body(idx_hbm, x_hbm, out_hbm, idx_v, x_v, out_v):
    wid = lax.axis_index("s") * _NC + lax.axis_index("c")
    base = wid * _B_PER_W
    pltpu.sync_copy(idx_hbm.at[pl.ds(base, _B_PER_W)], idx_v)
    pltpu.sync_copy(x_hbm.at[pl.ds(base, _B_PER_W)], x_v)
    rows0 = lax.iota(jnp.int32, _L)
    col_bases = []
    row_vecs = []
    for rc in range(_N_CHUNK):
        row_vecs.append(rows0 + rc * _L)
        col_bases.append(idx_v[pl.ds(rc * _L, _L)] * OUT_DIM)

    def jbody(j, carry):
        j_splat = jnp.full((_L,), j, jnp.int32)
        for rc in range(_N_CHUNK):
            vals = plsc.load_gather(x_v, [row_vecs[rc], col_bases[rc] + j])
            plsc.store_scatter(out_v, [row_vecs[rc], j_splat], vals)
        return carry

    lax.fori_loop(0, OUT_DIM, jbody, jnp.int32(0))
    pltpu.sync_copy(out_v, out_hbm.at[pl.ds(base, _B_PER_W)])


@jax.jit
def kernel(full_input, indices):
    idx_flat = indices.reshape(BATCH)
    run = functools.partial(
        pl.kernel,
        mesh=plsc.VectorSubcoreMesh(
            core_axis_name="c", subcore_axis_name="s", num_cores=_NC),
        out_type=jax.ShapeDtypeStruct((BATCH, OUT_DIM), jnp.float32),
        scratch_types=[
            pltpu.VMEM((_B_PER_W,), jnp.int32),
            pltpu.VMEM((_B_PER_W, WIDTH), jnp.float32),
            pltpu.VMEM((_B_PER_W, OUT_DIM), jnp.float32),
        ],
        compiler_params=pltpu.CompilerParams(
            disable_bounds_checks=True,
            disable_semaphore_checks=True,
            skip_device_barrier=True,
            needs_layout_passes=False,
        ),
    )(_mux_body)
    return run(idx_flat, full_input)


# speedup vs baseline: 1.0239x; 1.0239x over previous
import functools

import jax
import jax.numpy as jnp
from jax import lax
from jax.experimental import pallas as pl
from jax.experimental.pallas import tpu as pltpu
from jax.experimental.pallas import tpu_sc as plsc

OUT_DIM = 64
N_CTRL = 16
BATCH = 1024
WIDTH = OUT_DIM * N_CTRL

_INFO = plsc.get_sparse_core_info()
_NC = 1
_NS = _INFO.num_subcores
_NW = _NC * _NS
_L = _INFO.num_lanes
_B_PER_W = BATCH // _NW
_N_CHUNK = _B_PER_W // _L


def _mux_body(idx_hbm, x_hbm, out_hbm, idx_v, y_v, out_v, sem):
    wid = lax.axis_index("s") * _NC + lax.axis_index("c")
    base = wid * _B_PER_W
    pltpu.sync_copy(idx_hbm.at[pl.ds(base, _B_PER_W)], idx_v)
    iota16 = lax.iota(jnp.int32, _L)

    cs = []
    for rc in range(_N_CHUNK):
        idx16 = idx_v[pl.ds(rc * _L, _L)]
        for i in range(_L):
            sel = jnp.where(iota16 == i, idx16, 0)
            cs.append(lax.reduce_max(sel, (0,)))

    for rc in range(_N_CHUNK):
        for i in range(_L):
            r = rc * _L + i
            j = cs[r] // 2
            pltpu.async_copy(
                x_hbm.at[base + r, pl.ds(j * 2 * OUT_DIM, 2 * OUT_DIM)],
                y_v.at[r], sem)
        for i in range(_L):
            r = rc * _L + i
            pltpu.make_async_copy(
                x_hbm.at[base + r, pl.ds(0, 2 * OUT_DIM)],
                y_v.at[r], sem).wait()

    for r in range(_B_PER_W):
        off = (cs[r] - (cs[r] // 2) * 2) * OUT_DIM
        for k in range(OUT_DIM // _L):
            out_v[r, pl.ds(k * _L, _L)] = y_v[r, pl.ds(off + k * _L, _L)]

    pltpu.sync_copy(out_v, out_hbm.at[pl.ds(base, _B_PER_W)])


@jax.jit
def kernel(full_input, indices):
    idx_flat = indices.reshape(BATCH)
    run = functools.partial(
        pl.kernel,
        mesh=plsc.VectorSubcoreMesh(
            core_axis_name="c", subcore_axis_name="s", num_cores=_NC),
        out_type=jax.ShapeDtypeStruct((BATCH, OUT_DIM), jnp.float32),
        scratch_types=[
            pltpu.VMEM((_B_PER_W,), jnp.int32),
            pltpu.VMEM((_B_PER_W, 2 * OUT_DIM), jnp.float32),
            pltpu.VMEM((_B_PER_W, OUT_DIM), jnp.float32),
            pltpu.SemaphoreType.DMA,
        ],
        compiler_params=pltpu.CompilerParams(
            disable_bounds_checks=True,
            disable_semaphore_checks=True,
            skip_device_barrier=True,
            needs_layout_passes=False,
        ),
    )(_mux_body)
    return run(idx_flat, full_input)


# speedup vs baseline: 1.1191x; 1.0930x over previous
import functools

import jax
import jax.numpy as jnp
from jax import lax
from jax.experimental import pallas as pl
from jax.experimental.pallas import tpu as pltpu
from jax.experimental.pallas import tpu_sc as plsc

OUT_DIM = 64
N_CTRL = 16
BATCH = 1024
WIDTH = OUT_DIM * N_CTRL

_INFO = plsc.get_sparse_core_info()
_NC = 1
_NS = _INFO.num_subcores
_NW = _NC * _NS
_L = _INFO.num_lanes
_B_PER_W = BATCH // _NW


def _mux_body(idx_hbm, x_hbm, out_hbm, idx_v, x_v, out_v, sem):
    wid = lax.axis_index("s") * _NC + lax.axis_index("c")
    base = wid * _B_PER_W
    pltpu.async_copy(x_hbm.at[pl.ds(base, _B_PER_W)], x_v, sem)
    pltpu.sync_copy(idx_hbm.at[pl.ds(base, _B_PER_W)], idx_v)
    zeros16 = jnp.zeros((_L,), jnp.int32)
    col_offs = [k * _L + lax.iota(jnp.int32, _L) for k in range(OUT_DIM // _L)]
    pltpu.make_async_copy(x_hbm.at[pl.ds(base, _B_PER_W)], x_v, sem).wait()

    def rbody(i, carry):
        i16 = jnp.full((_L,), i, jnp.int32)
        c16 = plsc.load_gather(idx_v, [i16, zeros16]) * OUT_DIM
        for k in range(OUT_DIM // _L):
            vals = plsc.load_gather(x_v, [i16, c16 + col_offs[k]])
            out_v[i, pl.ds(k * _L, _L)] = vals
        return carry

    lax.fori_loop(0, _B_PER_W, rbody, jnp.int32(0))
    pltpu.sync_copy(out_v, out_hbm.at[pl.ds(base, _B_PER_W)])


@jax.jit
def kernel(full_input, indices):
    run = functools.partial(
        pl.kernel,
        mesh=plsc.VectorSubcoreMesh(
            core_axis_name="c", subcore_axis_name="s", num_cores=_NC),
        out_type=jax.ShapeDtypeStruct((BATCH, OUT_DIM), jnp.float32),
        scratch_types=[
            pltpu.VMEM((_B_PER_W, 1), jnp.int32),
            pltpu.VMEM((_B_PER_W, WIDTH), jnp.float32),
            pltpu.VMEM((_B_PER_W, OUT_DIM), jnp.float32),
            pltpu.SemaphoreType.DMA,
        ],
        compiler_params=pltpu.CompilerParams(
            disable_bounds_checks=True,
            disable_semaphore_checks=True,
            skip_device_barrier=True,
            needs_layout_passes=False,
        ),
    )(_mux_body)
    return run(indices, full_input)


# speedup vs baseline: 1.1672x; 1.0430x over previous
import functools

import jax
import jax.numpy as jnp
from jax import lax
from jax.experimental import pallas as pl
from jax.experimental.pallas import tpu as pltpu
from jax.experimental.pallas import tpu_sc as plsc

OUT_DIM = 64
N_CTRL = 16
BATCH = 1024
WIDTH = OUT_DIM * N_CTRL

_INFO = plsc.get_sparse_core_info()
_NC = _INFO.num_cores
_NS = _INFO.num_subcores
_NW = _NC * _NS
_L = _INFO.num_lanes
_B_PER_W = BATCH // _NW


def _mux_body(idx_hbm, x_hbm, out_hbm, idx_v, x_v, out_v, sem):
    wid = lax.axis_index("s") * _NC + lax.axis_index("c")
    base = wid * _B_PER_W
    pltpu.async_copy(x_hbm.at[pl.ds(base, _B_PER_W)], x_v, sem)
    pltpu.sync_copy(idx_hbm.at[pl.ds(base, _B_PER_W)], idx_v)
    col_offs = [k * _L + lax.iota(jnp.int32, _L) for k in range(OUT_DIM // _L)]
    pltpu.make_async_copy(x_hbm.at[pl.ds(base, _B_PER_W)], x_v, sem).wait()

    def rbody(i, carry):
        i16 = jnp.full((_L,), i, jnp.int32)
        c16 = plsc.load_gather(idx_v, [i16]) * OUT_DIM
        for k in range(OUT_DIM // _L):
            vals = plsc.load_gather(x_v, [i16, c16 + col_offs[k]])
            out_v[i, pl.ds(k * _L, _L)] = vals
        return carry

    lax.fori_loop(0, _B_PER_W, rbody, jnp.int32(0))
    pltpu.sync_copy(out_v, out_hbm.at[pl.ds(base, _B_PER_W)])


@jax.jit
def kernel(full_input, indices):
    run = functools.partial(
        pl.kernel,
        mesh=plsc.VectorSubcoreMesh(
            core_axis_name="c", subcore_axis_name="s", num_cores=_NC),
        out_type=jax.ShapeDtypeStruct((BATCH, OUT_DIM), jnp.float32),
        scratch_types=[
            pltpu.VMEM((_B_PER_W,), jnp.int32),
            pltpu.VMEM((_B_PER_W, WIDTH), jnp.float32),
            pltpu.VMEM((_B_PER_W, OUT_DIM), jnp.float32),
            pltpu.SemaphoreType.DMA,
        ],
        compiler_params=pltpu.CompilerParams(
            disable_bounds_checks=True,
            disable_semaphore_checks=True,
            skip_device_barrier=True,
            needs_layout_passes=False,
        ),
    )(_mux_body)
    return run(indices.reshape(BATCH), full_input)


# speedup vs baseline: 1.1874x; 1.0173x over previous
import functools

import jax
import jax.numpy as jnp
from jax import lax
from jax.experimental import pallas as pl
from jax.experimental.pallas import tpu as pltpu
from jax.experimental.pallas import tpu_sc as plsc

OUT_DIM = 64
N_CTRL = 16
BATCH = 1024
WIDTH = OUT_DIM * N_CTRL

_INFO = plsc.get_sparse_core_info()
_NC = _INFO.num_cores
_NS = _INFO.num_subcores
_NW = _NC * _NS
_L = _INFO.num_lanes
_B_PER_W = BATCH // _NW


def _mux_body(idx_hbm, x_hbm, out_hbm, idx_v, x_v, out_v, sem):
    wid = lax.axis_index("s") * _NC + lax.axis_index("c")
    base = wid * _B_PER_W
    pltpu.async_copy(x_hbm.at[pl.ds(base, _B_PER_W)], x_v, sem)
    pltpu.sync_copy(idx_hbm.at[pl.ds(base, _B_PER_W)], idx_v)
    col_offs = [k * _L + lax.iota(jnp.int32, _L) for k in range(OUT_DIM // _L)]
    pltpu.make_async_copy(x_hbm.at[pl.ds(base, _B_PER_W)], x_v, sem).wait()

    @plsc.parallel_loop(0, _B_PER_W, 1, unroll=4)
    def _rbody(i):
        i16 = jnp.full((_L,), i, jnp.int32)
        c16 = plsc.load_gather(idx_v, [i16]) * OUT_DIM
        for k in range(OUT_DIM // _L):
            vals = plsc.load_gather(x_v, [i16, c16 + col_offs[k]])
            out_v[i, pl.ds(k * _L, _L)] = vals
    pltpu.sync_copy(out_v, out_hbm.at[pl.ds(base, _B_PER_W)])


@jax.jit
def kernel(full_input, indices):
    run = functools.partial(
        pl.kernel,
        mesh=plsc.VectorSubcoreMesh(
            core_axis_name="c", subcore_axis_name="s", num_cores=_NC),
        out_type=jax.ShapeDtypeStruct((BATCH, OUT_DIM), jnp.float32),
        scratch_types=[
            pltpu.VMEM((_B_PER_W,), jnp.int32),
            pltpu.VMEM((_B_PER_W, WIDTH), jnp.float32),
            pltpu.VMEM((_B_PER_W, OUT_DIM), jnp.float32),
            pltpu.SemaphoreType.DMA,
        ],
        compiler_params=pltpu.CompilerParams(
            disable_bounds_checks=True,
            disable_semaphore_checks=True,
            skip_device_barrier=True,
            needs_layout_passes=False,
        ),
    )(_mux_body)
    return run(indices.reshape(BATCH), full_input)


# speedup vs baseline: 1.1990x; 1.0098x over previous
import functools

import jax
import jax.numpy as jnp
from jax import lax
from jax.experimental import pallas as pl
from jax.experimental.pallas import tpu as pltpu
from jax.experimental.pallas import tpu_sc as plsc

OUT_DIM = 64
N_CTRL = 16
BATCH = 1024
WIDTH = OUT_DIM * N_CTRL

_INFO = plsc.get_sparse_core_info()
_NC = 1
_NS = _INFO.num_subcores
_NW = _NC * _NS
_L = _INFO.num_lanes
_B_PER_W = BATCH // _NW


def _mux_body(idx_hbm, x_hbm, out_hbm, idx_v, x_v, out_v, sem):
    wid = lax.axis_index("s") * _NC + lax.axis_index("c")
    base = wid * _B_PER_W
    pltpu.async_copy(x_hbm.at[pl.ds(base, _B_PER_W)], x_v, sem)
    pltpu.sync_copy(idx_hbm.at[pl.ds(base, _B_PER_W)], idx_v)
    col_offs = [k * _L + lax.iota(jnp.int32, _L) for k in range(OUT_DIM // _L)]
    pltpu.make_async_copy(x_hbm.at[pl.ds(base, _B_PER_W)], x_v, sem).wait()

    @plsc.parallel_loop(0, _B_PER_W, 1, unroll=4)
    def _rbody(i):
        i16 = jnp.full((_L,), i, jnp.int32)
        c16 = plsc.load_gather(idx_v, [i16]) * OUT_DIM
        for k in range(OUT_DIM // _L):
            vals = plsc.load_gather(x_v, [i16, c16 + col_offs[k]])
            out_v[i, pl.ds(k * _L, _L)] = vals
    pltpu.sync_copy(out_v, out_hbm.at[pl.ds(base, _B_PER_W)])


@jax.jit
def kernel(full_input, indices):
    run = functools.partial(
        pl.kernel,
        mesh=plsc.VectorSubcoreMesh(
            core_axis_name="c", subcore_axis_name="s", num_cores=_NC),
        out_type=jax.ShapeDtypeStruct((BATCH, OUT_DIM), jnp.float32),
        scratch_types=[
            pltpu.VMEM((_B_PER_W,), jnp.int32),
            pltpu.VMEM((_B_PER_W, WIDTH), jnp.float32),
            pltpu.VMEM((_B_PER_W, OUT_DIM), jnp.float32),
            pltpu.SemaphoreType.DMA,
        ],
        compiler_params=pltpu.CompilerParams(
            disable_bounds_checks=True,
            disable_semaphore_checks=True,
            skip_device_barrier=True,
            needs_layout_passes=False,
        ),
    )(_mux_body)
    return run(indices.reshape(BATCH), full_input)
